# baseline (speedup 1.0000x reference)
import functools

import jax
import jax.numpy as jnp
from jax import lax
from jax.experimental import pallas as pl
from jax.experimental.pallas import tpu as pltpu
from jax.experimental.pallas import tpu_sc as plsc

_H = 32
_W = 32
_D = 768
_SEQ = _H * _W
_B = 64
_L = 16
_NC = 2
_DC = _D // _L
_SLAB = _W * _D
_CH = _W * _DC
_NBUF = 4


def _sc_body(x_hbm, row_hbm, colf_hbm, out_hbm,
             b0, b1, b2, b3, pe_v, row_v,
             si0, si1, si2, si3, so0, so1, so2, so3, sc_sem, sr_sem):
    wid = lax.axis_index("s") * _NC + lax.axis_index("c")
    base = wid * _SLAB
    bufs = (b0, b1, b2, b3)
    sin = (si0, si1, si2, si3)
    sout = (so0, so1, so2, so3)

    def in_copy(b, nb):
        return pltpu.make_async_copy(
            x_hbm.at[pl.ds(b * _SEQ * _D + base, _SLAB)], bufs[nb], sin[nb])

    def out_copy(b, nb):
        return pltpu.make_async_copy(
            bufs[nb], out_hbm.at[pl.ds(b * _SEQ * _D + base, _SLAB)],
            sout[nb])

    in_copy(0, 0).start()
    in_copy(1, 1).start()
    in_copy(2, 2).start()
    h_c = pltpu.async_copy(colf_hbm, pe_v, sc_sem)
    h_r = pltpu.async_copy(row_hbm.at[wid], row_v, sr_sem)
    h_c.wait()
    h_r.wait()

    @plsc.parallel_loop(0, _CH, unroll=8)
    def _(i):
        ci = lax.rem(i, _DC)
        pe_v[pl.ds(i * _L, _L)] = (
            pe_v[pl.ds(i * _L, _L)] + row_v[pl.ds(ci * _L, _L)]
        )

    def g_body(g, carry):
        for nb in range(_NBUF):
            b = g * _NBUF + nb
            buf = bufs[nb]
            in_copy(b, nb).wait()

            @plsc.parallel_loop(0, _CH, unroll=8)
            def _(i):
                buf[pl.ds(i * _L, _L)] = (
                    buf[pl.ds(i * _L, _L)] + pe_v[pl.ds(i * _L, _L)]
                )

            out_copy(b, nb).start()
            rb = b + _NBUF - 1
            prev = (nb + _NBUF - 1) % _NBUF

            @pl.when(rb < _B)
            def _():
                @pl.when(rb >= _NBUF)
                def _():
                    out_copy(rb - _NBUF, prev).wait()

                in_copy(rb, prev).start()

        return carry

    lax.fori_loop(0, _B // _NBUF, g_body, 0)
    for nb in range(_NBUF):
        out_copy(_B - _NBUF + nb, nb).wait()


_sc_full = functools.partial(
    pl.kernel,
    out_type=jax.ShapeDtypeStruct((_B * _SEQ * _D,), jnp.float32),
    mesh=plsc.VectorSubcoreMesh(core_axis_name="c", subcore_axis_name="s"),
    scratch_types=(
        [pltpu.VMEM((_SLAB,), jnp.float32) for _ in range(_NBUF)]
        + [pltpu.VMEM((_SLAB,), jnp.float32), pltpu.VMEM((_D,), jnp.float32)]
        + [pltpu.SemaphoreType.DMA] * 10
    ),
)(_sc_body)


def kernel(x, pos_row, pos_col):
    B, SEQ, D = x.shape
    out = _sc_full(x.reshape(-1), pos_row, pos_col.reshape(-1))
    return out.reshape(B, SEQ, D)


# speedup vs baseline: 3.6033x; 3.6033x over previous
import functools

import jax
import jax.numpy as jnp
from jax import lax
from jax.experimental import pallas as pl
from jax.experimental.pallas import tpu as pltpu
from jax.experimental.pallas import tpu_sc as plsc

_H = 32
_W = 32
_D = 768
_SEQ = _H * _W
_NB = 2
_L = 16
_NC = 2
_DC = _D // _L
_SLAB = _W * _D


_Q = 4
_QCH = _W * _DC // _Q
_QEL = _QCH * _L


def _pe_sc_body(row_hbm, colf_hbm, out_hbm, row_v, col_v, out_v,
                sem_r, sem_c, sem_o):
    wid = lax.axis_index("s") * _NC + lax.axis_index("c")
    h_r = pltpu.async_copy(row_hbm.at[wid], row_v, sem_r)
    h_c = pltpu.async_copy(colf_hbm, col_v, sem_c)
    h_r.wait()
    h_c.wait()

    outs = []
    for q in range(_Q):

        @plsc.parallel_loop(q * _QCH, (q + 1) * _QCH, unroll=8)
        def _(i):
            ci = lax.rem(i, _DC)
            out_v[pl.ds(i * _L, _L)] = (
                col_v[pl.ds(i * _L, _L)] + row_v[pl.ds(ci * _L, _L)]
            )

        outs.append(pltpu.async_copy(
            out_v.at[pl.ds(q * _QEL, _QEL)],
            out_hbm.at[pl.ds(wid * _SLAB + q * _QEL, _QEL)],
            sem_o,
        ))
    for h in outs:
        h.wait()


_pe_sc = functools.partial(
    pl.kernel,
    out_type=jax.ShapeDtypeStruct((_SEQ * _D,), jnp.float32),
    mesh=plsc.VectorSubcoreMesh(core_axis_name="c", subcore_axis_name="s"),
    scratch_types=[
        pltpu.VMEM((_D,), jnp.float32),
        pltpu.VMEM((_SLAB,), jnp.float32),
        pltpu.VMEM((_SLAB,), jnp.float32),
        pltpu.SemaphoreType.DMA,
        pltpu.SemaphoreType.DMA,
        pltpu.SemaphoreType.DMA,
    ],
)(_pe_sc_body)


def _add_body(x_ref, pe_ref, o_ref):
    o_ref[...] = x_ref[...] + pe_ref[...][None]


def kernel(x, pos_row, pos_col):
    B, SEQ, D = x.shape
    pe = _pe_sc(pos_row, pos_col.reshape(-1)).reshape(SEQ, D)
    out = pl.pallas_call(
        _add_body,
        grid=(B // _NB,),
        in_specs=[
            pl.BlockSpec((_NB, SEQ, D), lambda b: (b, 0, 0)),
            pl.BlockSpec((SEQ, D), lambda b: (0, 0)),
        ],
        out_specs=pl.BlockSpec((_NB, SEQ, D), lambda b: (b, 0, 0)),
        out_shape=jax.ShapeDtypeStruct((B, SEQ, D), x.dtype),
    )(x, pe)
    return out


# speedup vs baseline: 3.6846x; 1.0226x over previous
import functools

import jax
import jax.numpy as jnp
from jax import lax
from jax.experimental import pallas as pl
from jax.experimental.pallas import tpu as pltpu
from jax.experimental.pallas import tpu_sc as plsc

_H = 32
_W = 32
_D = 768
_SEQ = _H * _W
_NB = 4
_L = 16
_NC = 2
_DC = _D // _L
_Q = 4
_QW = _W // _Q
_QCH = _QW * _DC


def _pe_sc_body(row_hbm, col_hbm, out_hbm, row_v, col_v, out_v,
                sem_r, sem_c, sem_o):
    wid = lax.axis_index("s") * _NC + lax.axis_index("c")
    h_r = pltpu.async_copy(row_hbm.at[wid], row_v, sem_r)
    h_c = pltpu.async_copy(col_hbm, col_v, sem_c)
    h_r.wait()
    h_c.wait()

    outs = []
    for q in range(_Q):

        @plsc.parallel_loop(q * _QCH, (q + 1) * _QCH, unroll=8)
        def _(i):
            w = i // _DC
            ci = lax.rem(i, _DC)
            out_v[w, pl.ds(ci * _L, _L)] = (
                col_v[w, pl.ds(ci * _L, _L)] + row_v[pl.ds(ci * _L, _L)]
            )

        outs.append(pltpu.async_copy(
            out_v.at[pl.ds(q * _QW, _QW)],
            out_hbm.at[pl.ds(wid * _W + q * _QW, _QW)],
            sem_o,
        ))
    for h in outs:
        h.wait()


_pe_sc = functools.partial(
    pl.kernel,
    out_type=jax.ShapeDtypeStruct((_SEQ, _D), jnp.float32),
    mesh=plsc.VectorSubcoreMesh(core_axis_name="c", subcore_axis_name="s"),
    scratch_types=[
        pltpu.VMEM((_D,), jnp.float32),
        pltpu.VMEM((_W, _D), jnp.float32),
        pltpu.VMEM((_W, _D), jnp.float32),
        pltpu.SemaphoreType.DMA,
        pltpu.SemaphoreType.DMA,
        pltpu.SemaphoreType.DMA,
    ],
)(_pe_sc_body)


def _add_body(x_ref, pe_ref, o_ref):
    o_ref[...] = x_ref[...] + pe_ref[...][None]


def kernel(x, pos_row, pos_col):
    B, SEQ, D = x.shape
    pe = _pe_sc(pos_row, pos_col)
    out = pl.pallas_call(
        _add_body,
        grid=(B // _NB,),
        in_specs=[
            pl.BlockSpec((_NB, SEQ, D), lambda b: (b, 0, 0)),
            pl.BlockSpec((SEQ, D), lambda b: (0, 0)),
        ],
        out_specs=pl.BlockSpec((_NB, SEQ, D), lambda b: (b, 0, 0)),
        out_shape=jax.ShapeDtypeStruct((B, SEQ, D), x.dtype),
    )(x, pe)
    return out
